# baseline (speedup 1.0000x reference)
import jax
import jax.numpy as jnp
from jax import lax
from jax.experimental import pallas as pl
from jax.experimental.pallas import tpu as pltpu
from jax.experimental.pallas import tpu_sc as plsc

B, S, F = 16, 2048, 4
H = 768
HP = H // 2
VOCAB = 30522
MAX_POS = 8192
EPS = 1e-12

L = 16
KCH = H // L
PCH = HP // L
NW = 32
NROWS = B * S
ROWS_PER_W = NROWS // NW
BATCH_ROWS = 8
PIECES = BATCH_ROWS * F
NBATCH = ROWS_PER_W // BATCH_ROWS
CHUNK = ROWS_PER_W * F
SENT_PIECES = S * F


def _rsqrt(x):
    xi = lax.bitcast_convert_type(x, jnp.int32)
    yi = jnp.int32(0x5F3759DF) - lax.shift_right_logical(xi, 1)
    y = lax.bitcast_convert_type(yi, jnp.float32)
    for _ in range(4):
        y = y * (1.5 - 0.5 * x * y * y)
    return y


def _sc_body(ids_hbm, wt_hbm, pt_hbm, lnw_hbm, lnb_hbm, out_hbm,
             idsbuf, idxw0, idxp0, idxw1, idxp1,
             wbuf0, pbuf0, wbuf1, pbuf1,
             accbuf, xbuf, sbuf, qbuf, outbuf0, outbuf1, lnwbuf, lnbbuf,
             gsem0, gsem1, osem0, osem1):
    wid = lax.axis_index("s") * 2 + lax.axis_index("c")
    sent = wid // 2
    half = wid % 2
    iota16 = lax.iota(jnp.int32, 16)
    zv = jnp.zeros((L,), jnp.float32)

    gslot = [(idxw0, idxp0, wbuf0, pbuf0, gsem0),
             (idxw1, idxp1, wbuf1, pbuf1, gsem1)]
    oslot = [(outbuf0, osem0), (outbuf1, osem1)]

    pltpu.sync_copy(lnw_hbm, lnwbuf)
    pltpu.sync_copy(lnb_hbm, lnbbuf)

    first_half_off = sent * SENT_PIECES
    pltpu.sync_copy(ids_hbm.at[pl.ds(first_half_off, CHUNK)], idsbuf)

    def _count_step(j, cnt):
        v = idsbuf[pl.ds(j * L, L)]
        return cnt + jnp.where(v != 0, 1, 0)

    cnt_v = lax.fori_loop(0, CHUNK // L, _count_step,
                          jnp.zeros((L,), jnp.int32), unroll=8)
    base0 = half * jnp.sum(cnt_v)

    my_off = wid * CHUNK
    pltpu.sync_copy(ids_hbm.at[pl.ds(my_off, CHUNK)], idsbuf)

    def _masks(i):
        off = i * PIECES
        ids0 = idsbuf[pl.ds(off, L)]
        ids1 = idsbuf[pl.ds(off + L, L)]
        return (jnp.where(ids0 != 0, 1, 0), jnp.where(ids1 != 0, 1, 0),
                ids0, ids1)

    def _fire(i, base, slot):
        idxw, idxp, wbuf, pbuf, gsem = gslot[slot]
        mi0, mi1, ids0, ids1 = _masks(i)
        c0 = plsc.cumsum(mi0)
        c1 = plsc.cumsum(mi1)
        t0 = jnp.sum(mi0)
        t1 = jnp.sum(mi1)
        pos0 = jnp.clip(base + c0 - 1, 0, MAX_POS - 1)
        pos1 = jnp.clip(base + t0 + c1 - 1, 0, MAX_POS - 1)
        idxw[pl.ds(0, L)] = ids0
        idxw[pl.ds(L, L)] = ids1
        idxp[pl.ds(0, L)] = pos0
        idxp[pl.ds(L, L)] = pos1
        pltpu.async_copy(wt_hbm.at[idxw], wbuf, gsem)
        pltpu.async_copy(pt_hbm.at[idxp], pbuf, gsem)
        return base + t0 + t1

    SQG = 17 * L

    def _splat(vec, lane):
        return jnp.take_along_axis(vec, jnp.full((L,), lane, jnp.int32),
                                   axis=0)

    def _unpk(v):
        return plsc.unpack(plsc.bitcast(v, jnp.bfloat16),
                           format=plsc.PackFormat.INTERLEAVED)

    def _process(i, slot):
        idxw, idxp, wbuf, pbuf, gsem = gslot[slot]
        outbuf, osem = oslot[slot]

        def _zero(j, _):
            accbuf[pl.ds(j * L, L)] = zv
            return 0

        lax.fori_loop(0, BATCH_ROWS * KCH, _zero, 0, unroll=8)

        pltpu.make_async_copy(wt_hbm.at[idxw], wbuf, gsem).wait()
        pltpu.make_async_copy(pt_hbm.at[idxp], pbuf, gsem).wait()

        mi0, mi1, _, _ = _masks(i)

        iota17 = iota16 * 17

        def _stats(p, _):
            xoff = p * H

            def _stat(c, carry):
                s0, s1, q0, q1 = carry
                wlo, whi = _unpk(wbuf[p, pl.ds(c * L, L)])
                plo, phi = _unpk(pbuf[p, pl.ds(c * L, L)])
                x0 = wlo + plo
                x1 = whi + phi
                xbuf[pl.ds(xoff + c * L, L)] = x0
                xbuf[pl.ds(xoff + (c + PCH) * L, L)] = x1
                return (s0 + x0, s1 + x1, q0 + x0 * x0, q1 + x1 * x1)

            s0, s1, q0, q1 = lax.fori_loop(0, PCH, _stat, (zv,) * 4,
                                           unroll=8)
            s_v = s0 + s1
            q_v = q0 + q1
            g = p // L
            off = iota17 + (p - g * L + g * SQG)
            plsc.store_scatter(sbuf, [off], s_v)
            plsc.store_scatter(qbuf, [off], q_v)
            return 0

        lax.fori_loop(0, PIECES, _stats, 0)

        abs_ = []
        for g, mi in ((0, mi0), (1, mi1)):
            tot_s = sbuf[pl.ds(g * SQG, L)]
            tot_q = qbuf[pl.ds(g * SQG, L)]
            for c in range(1, L):
                tot_s = tot_s + sbuf[pl.ds(g * SQG + c * 17, L)]
                tot_q = tot_q + qbuf[pl.ds(g * SQG + c * 17, L)]
            mu_v = tot_s * (1.0 / H)
            var_v = tot_q * (1.0 / H) - mu_v * mu_v
            rstd_v = _rsqrt(var_v + EPS)
            a_v = rstd_v * mi.astype(jnp.float32)
            abs_.append((a_v, -mu_v * a_v))

        (a0, b0), (a1, b1) = abs_

        def _piece(p, _):
            pm = p & (L - 1)
            av = jnp.where(p < L, a0, a1)
            bv = jnp.where(p < L, b0, b1)
            a_bc = _splat(av, pm)
            b_bc = _splat(bv, pm)
            rowoff = (p // F) * H
            xoff = p * H

            def _norm(k, _):
                plsc.addupdate(accbuf.at[pl.ds(rowoff + k * L, L)],
                               xbuf[pl.ds(xoff + k * L, L)] * a_bc + b_bc)
                return 0

            lax.fori_loop(0, KCH, _norm, 0, unroll=8)
            return 0

        lax.fori_loop(0, PIECES, _piece, 0)

        @pl.when(i >= 2)
        def _():
            pltpu.make_async_copy(
                outbuf, out_hbm.at[pl.ds(0, BATCH_ROWS)], osem).wait()

        perm1 = iota16 ^ 1
        perm2 = iota16 ^ 2
        invs, anys = [], []
        for mi in (mi0, mi1):
            r1 = mi + jnp.take_along_axis(mi, perm1, axis=0)
            cnt4 = r1 + jnp.take_along_axis(r1, perm2, axis=0)
            invs.append(jnp.where(cnt4 <= 1, 1.0,
                                  jnp.where(cnt4 == 2, 0.5,
                                            jnp.where(cnt4 == 3, 1.0 / 3.0,
                                                      0.25))))
            anys.append(jnp.where(cnt4 > 0, 1.0, 0.0))

        for r in range(BATCH_ROWS):
            g = 0 if r < 4 else 1
            lane = (r % 4) * F
            inv_bc = _splat(invs[g], lane)
            any_bc = _splat(anys[g], lane)

            def _fin(k, _, r=r, inv_bc=inv_bc, any_bc=any_bc):
                o = accbuf[pl.ds(r * H + k * L, L)] * inv_bc
                o = o * lnwbuf[pl.ds(k * L, L)] \
                    + lnbbuf[pl.ds(k * L, L)] * any_bc
                outbuf[r, pl.ds(k * L, L)] = o
                return 0

            lax.fori_loop(0, KCH, _fin, 0, unroll=6)

        rowbase = wid * ROWS_PER_W + i * BATCH_ROWS
        pltpu.async_copy(outbuf, out_hbm.at[pl.ds(rowbase, BATCH_ROWS)], osem)

    base = _fire(0, base0, 0)

    def _pair(j, base):
        i0 = 2 * j
        base = _fire(i0 + 1, base, 1)
        _process(i0, 0)
        base = _fire(jnp.minimum(i0 + 2, NBATCH - 1), base, 0)
        _process(i0 + 1, 1)
        return base

    lax.fori_loop(0, NBATCH // 2, _pair, base)

    pltpu.make_async_copy(wt_hbm.at[idxw0], wbuf0, gsem0).wait()
    pltpu.make_async_copy(pt_hbm.at[idxp0], pbuf0, gsem0).wait()
    pltpu.make_async_copy(outbuf0, out_hbm.at[pl.ds(0, BATCH_ROWS)],
                          osem0).wait()
    pltpu.make_async_copy(outbuf1, out_hbm.at[pl.ds(0, BATCH_ROWS)],
                          osem1).wait()


def _pack_wt(x_ref, o_ref):
    e = x_ref[:, 0:HP]
    o = x_ref[:, HP:H]
    eu = lax.bitcast_convert_type(e.astype(jnp.bfloat16),
                                  jnp.uint16).astype(jnp.uint32)
    ou = lax.bitcast_convert_type(o.astype(jnp.bfloat16),
                                  jnp.uint16).astype(jnp.uint32)
    o_ref[...] = (eu | (ou << 16)).astype(jnp.int32)


def _pack_pt(pt_ref, tt_ref, o_ref):
    y = pt_ref[...] + tt_ref[...]
    e = y[:, 0:HP]
    o = y[:, HP:H]
    eu = lax.bitcast_convert_type(e.astype(jnp.bfloat16),
                                  jnp.uint16).astype(jnp.uint32)
    ou = lax.bitcast_convert_type(o.astype(jnp.bfloat16),
                                  jnp.uint16).astype(jnp.uint32)
    o_ref[...] = (eu | (ou << 16)).astype(jnp.int32)


def kernel(words, word_table, pos_table, tt_table, ln_w, ln_b):
    WBLK = 728
    wt_packed = pl.pallas_call(
        _pack_wt,
        grid=(42,),
        in_specs=[pl.BlockSpec((WBLK, H), lambda i: (i, 0))],
        out_specs=pl.BlockSpec((WBLK, HP), lambda i: (i, 0)),
        out_shape=jax.ShapeDtypeStruct((VOCAB, HP), jnp.int32),
    )(word_table)

    pt_packed = pl.pallas_call(
        _pack_pt,
        grid=(8,),
        in_specs=[
            pl.BlockSpec((MAX_POS // 8, H), lambda i: (i, 0)),
            pl.BlockSpec((1, H), lambda i: (0, 0)),
        ],
        out_specs=pl.BlockSpec((MAX_POS // 8, HP), lambda i: (i, 0)),
        out_shape=jax.ShapeDtypeStruct((MAX_POS, HP), jnp.int32),
    )(pos_table, tt_table[0:1])

    ids = words.reshape(NROWS * F)

    mesh = plsc.VectorSubcoreMesh(core_axis_name="c", subcore_axis_name="s")
    sc = pl.kernel(
        _sc_body,
        out_type=jax.ShapeDtypeStruct((NROWS, H), jnp.float32),
        mesh=mesh,
        compiler_params=pltpu.CompilerParams(needs_layout_passes=False),
        scratch_types=[
            pltpu.VMEM((CHUNK,), jnp.int32),
            pltpu.VMEM((PIECES,), jnp.int32),
            pltpu.VMEM((PIECES,), jnp.int32),
            pltpu.VMEM((PIECES,), jnp.int32),
            pltpu.VMEM((PIECES,), jnp.int32),
            pltpu.VMEM((PIECES, HP), jnp.int32),
            pltpu.VMEM((PIECES, HP), jnp.int32),
            pltpu.VMEM((PIECES, HP), jnp.int32),
            pltpu.VMEM((PIECES, HP), jnp.int32),
            pltpu.VMEM((BATCH_ROWS * H,), jnp.float32),
            pltpu.VMEM((PIECES * H,), jnp.float32),
            pltpu.VMEM((2 * 17 * L,), jnp.float32),
            pltpu.VMEM((2 * 17 * L,), jnp.float32),
            pltpu.VMEM((BATCH_ROWS, H), jnp.float32),
            pltpu.VMEM((BATCH_ROWS, H), jnp.float32),
            pltpu.VMEM((H,), jnp.float32),
            pltpu.VMEM((H,), jnp.float32),
            pltpu.SemaphoreType.DMA,
            pltpu.SemaphoreType.DMA,
            pltpu.SemaphoreType.DMA,
            pltpu.SemaphoreType.DMA,
        ],
    )
    out = sc(ids, wt_packed, pt_packed, ln_w, ln_b)
    return out.reshape(B, S, H)


# speedup vs baseline: 2.6339x; 2.6339x over previous
import jax
import jax.numpy as jnp
from jax import lax
from jax.experimental import pallas as pl
from jax.experimental.pallas import tpu as pltpu
from jax.experimental.pallas import tpu_sc as plsc

B, S, F = 16, 2048, 4
H = 768
HP = H // 2
VOCAB = 30522
MAX_POS = 8192
EPS = 1e-12

L = 16
KCH = H // L
PCH = HP // L
NW = 32
NROWS = B * S
ROWS_PER_W = NROWS // NW
BATCH_ROWS = 8
PIECES = BATCH_ROWS * F
NBATCH = ROWS_PER_W // BATCH_ROWS
CHUNK = ROWS_PER_W * F
SENT_PIECES = S * F


def _rsqrt(x):
    xi = lax.bitcast_convert_type(x, jnp.int32)
    yi = jnp.int32(0x5F3759DF) - lax.shift_right_logical(xi, 1)
    y = lax.bitcast_convert_type(yi, jnp.float32)
    for _ in range(4):
        y = y * (1.5 - 0.5 * x * y * y)
    return y


def _sc_body(ids_hbm, wt_hbm, pt_hbm, lnw_hbm, lnb_hbm, out_hbm,
             idsbuf, idxw0, idxp0, idxw1, idxp1,
             wbuf0, pbuf0, wbuf1, pbuf1,
             accbuf, xbuf, sbuf, qbuf, outbuf0, outbuf1, lnwbuf, lnbbuf,
             gsem0, gsem1, osem0, osem1):
    wid = lax.axis_index("s") * 2 + lax.axis_index("c")
    sent = wid // 2
    half = wid % 2
    iota16 = lax.iota(jnp.int32, 16)
    zv = jnp.zeros((L,), jnp.float32)

    gslot = [(idxw0, idxp0, wbuf0, pbuf0, gsem0),
             (idxw1, idxp1, wbuf1, pbuf1, gsem1)]
    oslot = [(outbuf0, osem0), (outbuf1, osem1)]

    pltpu.sync_copy(lnw_hbm, lnwbuf)
    pltpu.sync_copy(lnb_hbm, lnbbuf)

    first_half_off = sent * SENT_PIECES
    pltpu.sync_copy(ids_hbm.at[pl.ds(first_half_off, CHUNK)], idsbuf)

    def _count_step(j, cnt):
        v = idsbuf[pl.ds(j * L, L)]
        return cnt + jnp.where(v != 0, 1, 0)

    cnt_v = lax.fori_loop(0, CHUNK // L, _count_step,
                          jnp.zeros((L,), jnp.int32), unroll=8)
    base0 = half * jnp.sum(cnt_v)

    my_off = wid * CHUNK
    pltpu.sync_copy(ids_hbm.at[pl.ds(my_off, CHUNK)], idsbuf)

    def _masks(i):
        off = i * PIECES
        ids0 = idsbuf[pl.ds(off, L)]
        ids1 = idsbuf[pl.ds(off + L, L)]
        return (jnp.where(ids0 != 0, 1, 0), jnp.where(ids1 != 0, 1, 0),
                ids0, ids1)

    def _fire(i, base, slot):
        idxw, idxp, wbuf, pbuf, gsem = gslot[slot]
        mi0, mi1, ids0, ids1 = _masks(i)
        c0 = plsc.cumsum(mi0)
        c1 = plsc.cumsum(mi1)
        t0 = jnp.sum(mi0)
        t1 = jnp.sum(mi1)
        pos0 = jnp.clip(base + c0 - 1, 0, MAX_POS - 1)
        pos1 = jnp.clip(base + t0 + c1 - 1, 0, MAX_POS - 1)
        idxw[pl.ds(0, L)] = ids0
        idxw[pl.ds(L, L)] = ids1
        idxp[pl.ds(0, L)] = pos0
        idxp[pl.ds(L, L)] = pos1
        pltpu.async_copy(wt_hbm.at[idxw], wbuf, gsem)
        pltpu.async_copy(pt_hbm.at[idxp], pbuf, gsem)
        return base + t0 + t1

    SQG = 17 * L

    def _splat(vec, lane):
        return jnp.take_along_axis(vec, jnp.full((L,), lane, jnp.int32),
                                   axis=0)

    def _unpk(v):
        return plsc.unpack(plsc.bitcast(v, jnp.bfloat16),
                           format=plsc.PackFormat.INTERLEAVED)

    def _process(i, slot):
        idxw, idxp, wbuf, pbuf, gsem = gslot[slot]
        outbuf, osem = oslot[slot]

        def _zero(j, _):
            accbuf[pl.ds(j * L, L)] = zv
            return 0

        lax.fori_loop(0, BATCH_ROWS * KCH, _zero, 0, unroll=8)

        pltpu.make_async_copy(wt_hbm.at[idxw], wbuf, gsem).wait()
        pltpu.make_async_copy(pt_hbm.at[idxp], pbuf, gsem).wait()

        mi0, mi1, _, _ = _masks(i)

        iota17 = iota16 * 17

        def _stats(p, _):
            xoff = p * H
            GRP = 8

            def _stat(c8, carry):
                base = c8 * GRP
                wv = [wbuf[p, pl.ds((base + u) * L, L)] for u in range(GRP)]
                pv = [pbuf[p, pl.ds((base + u) * L, L)] for u in range(GRP)]
                accs = list(carry)
                for u in range(GRP):
                    wlo, whi = _unpk(wv[u])
                    plo, phi = _unpk(pv[u])
                    x0 = wlo + plo
                    x1 = whi + phi
                    xbuf[pl.ds(xoff + (base + u) * L, L)] = x0
                    xbuf[pl.ds(xoff + (base + u + PCH) * L, L)] = x1
                    j = u % 2
                    accs[j] = accs[j] + x0
                    accs[2 + j] = accs[2 + j] + x1
                    accs[4 + j] = accs[4 + j] + x0 * x0
                    accs[6 + j] = accs[6 + j] + x1 * x1
                return tuple(accs)

            acc = lax.fori_loop(0, PCH // GRP, _stat, (zv,) * 8)
            s_v = (acc[0] + acc[1]) + (acc[2] + acc[3])
            q_v = (acc[4] + acc[5]) + (acc[6] + acc[7])
            g = p // L
            off = iota17 + (p - g * L + g * SQG)
            plsc.store_scatter(sbuf, [off], s_v)
            plsc.store_scatter(qbuf, [off], q_v)
            return 0

        lax.fori_loop(0, PIECES, _stats, 0)

        abs_ = []
        for g, mi in ((0, mi0), (1, mi1)):
            tot_s = sbuf[pl.ds(g * SQG, L)]
            tot_q = qbuf[pl.ds(g * SQG, L)]
            for c in range(1, L):
                tot_s = tot_s + sbuf[pl.ds(g * SQG + c * 17, L)]
                tot_q = tot_q + qbuf[pl.ds(g * SQG + c * 17, L)]
            mu_v = tot_s * (1.0 / H)
            var_v = tot_q * (1.0 / H) - mu_v * mu_v
            rstd_v = _rsqrt(var_v + EPS)
            a_v = rstd_v * mi.astype(jnp.float32)
            abs_.append((a_v, -mu_v * a_v))

        (a0, b0), (a1, b1) = abs_

        def _piece(p, _):
            pm = p & (L - 1)
            av = jnp.where(p < L, a0, a1)
            bv = jnp.where(p < L, b0, b1)
            a_bc = _splat(av, pm)
            b_bc = _splat(bv, pm)
            rowoff = (p // F) * H
            xoff = p * H

            GRP = 8

            def _norm(k8, _):
                base = k8 * GRP
                xv = [xbuf[pl.ds(xoff + (base + u) * L, L)]
                      for u in range(GRP)]
                tv = [x * a_bc + b_bc for x in xv]
                for u in range(GRP):
                    plsc.addupdate(
                        accbuf.at[pl.ds(rowoff + (base + u) * L, L)], tv[u])
                return 0

            lax.fori_loop(0, KCH // GRP, _norm, 0)
            return 0

        lax.fori_loop(0, PIECES, _piece, 0)

        @pl.when(i >= 2)
        def _():
            pltpu.make_async_copy(
                outbuf, out_hbm.at[pl.ds(0, BATCH_ROWS)], osem).wait()

        perm1 = iota16 ^ 1
        perm2 = iota16 ^ 2
        invs, anys = [], []
        for mi in (mi0, mi1):
            r1 = mi + jnp.take_along_axis(mi, perm1, axis=0)
            cnt4 = r1 + jnp.take_along_axis(r1, perm2, axis=0)
            invs.append(jnp.where(cnt4 <= 1, 1.0,
                                  jnp.where(cnt4 == 2, 0.5,
                                            jnp.where(cnt4 == 3, 1.0 / 3.0,
                                                      0.25))))
            anys.append(jnp.where(cnt4 > 0, 1.0, 0.0))

        for r in range(BATCH_ROWS):
            g = 0 if r < 4 else 1
            lane = (r % 4) * F
            inv_bc = _splat(invs[g], lane)
            any_bc = _splat(anys[g], lane)

            def _fin(k6, _, r=r, inv_bc=inv_bc, any_bc=any_bc):
                base = k6 * 6
                av = [accbuf[pl.ds(r * H + (base + u) * L, L)]
                      for u in range(6)]
                lw = [lnwbuf[pl.ds((base + u) * L, L)] for u in range(6)]
                lb = [lnbbuf[pl.ds((base + u) * L, L)] for u in range(6)]
                for u in range(6):
                    o = av[u] * inv_bc * lw[u] + lb[u] * any_bc
                    outbuf[r, pl.ds((base + u) * L, L)] = o
                return 0

            lax.fori_loop(0, KCH // 6, _fin, 0)

        rowbase = wid * ROWS_PER_W + i * BATCH_ROWS
        pltpu.async_copy(outbuf, out_hbm.at[pl.ds(rowbase, BATCH_ROWS)], osem)

    base = _fire(0, base0, 0)

    def _pair(j, base):
        i0 = 2 * j
        base = _fire(i0 + 1, base, 1)
        _process(i0, 0)
        base = _fire(jnp.minimum(i0 + 2, NBATCH - 1), base, 0)
        _process(i0 + 1, 1)
        return base

    lax.fori_loop(0, NBATCH // 2, _pair, base)

    pltpu.make_async_copy(wt_hbm.at[idxw0], wbuf0, gsem0).wait()
    pltpu.make_async_copy(pt_hbm.at[idxp0], pbuf0, gsem0).wait()
    pltpu.make_async_copy(outbuf0, out_hbm.at[pl.ds(0, BATCH_ROWS)],
                          osem0).wait()
    pltpu.make_async_copy(outbuf1, out_hbm.at[pl.ds(0, BATCH_ROWS)],
                          osem1).wait()


def _pack_wt(x_ref, o_ref):
    e = x_ref[:, 0:HP]
    o = x_ref[:, HP:H]
    eu = lax.bitcast_convert_type(e.astype(jnp.bfloat16),
                                  jnp.uint16).astype(jnp.uint32)
    ou = lax.bitcast_convert_type(o.astype(jnp.bfloat16),
                                  jnp.uint16).astype(jnp.uint32)
    o_ref[...] = (eu | (ou << 16)).astype(jnp.int32)


def _pack_pt(pt_ref, tt_ref, o_ref):
    y = pt_ref[...] + tt_ref[...]
    e = y[:, 0:HP]
    o = y[:, HP:H]
    eu = lax.bitcast_convert_type(e.astype(jnp.bfloat16),
                                  jnp.uint16).astype(jnp.uint32)
    ou = lax.bitcast_convert_type(o.astype(jnp.bfloat16),
                                  jnp.uint16).astype(jnp.uint32)
    o_ref[...] = (eu | (ou << 16)).astype(jnp.int32)


def kernel(words, word_table, pos_table, tt_table, ln_w, ln_b):
    WBLK = 728
    wt_packed = pl.pallas_call(
        _pack_wt,
        grid=(42,),
        in_specs=[pl.BlockSpec((WBLK, H), lambda i: (i, 0))],
        out_specs=pl.BlockSpec((WBLK, HP), lambda i: (i, 0)),
        out_shape=jax.ShapeDtypeStruct((VOCAB, HP), jnp.int32),
    )(word_table)

    pt_packed = pl.pallas_call(
        _pack_pt,
        grid=(8,),
        in_specs=[
            pl.BlockSpec((MAX_POS // 8, H), lambda i: (i, 0)),
            pl.BlockSpec((1, H), lambda i: (0, 0)),
        ],
        out_specs=pl.BlockSpec((MAX_POS // 8, HP), lambda i: (i, 0)),
        out_shape=jax.ShapeDtypeStruct((MAX_POS, HP), jnp.int32),
    )(pos_table, tt_table[0:1])

    ids = words.reshape(NROWS * F)

    mesh = plsc.VectorSubcoreMesh(core_axis_name="c", subcore_axis_name="s")
    sc = pl.kernel(
        _sc_body,
        out_type=jax.ShapeDtypeStruct((NROWS, H), jnp.float32),
        mesh=mesh,
        compiler_params=pltpu.CompilerParams(needs_layout_passes=False),
        scratch_types=[
            pltpu.VMEM((CHUNK,), jnp.int32),
            pltpu.VMEM((PIECES,), jnp.int32),
            pltpu.VMEM((PIECES,), jnp.int32),
            pltpu.VMEM((PIECES,), jnp.int32),
            pltpu.VMEM((PIECES,), jnp.int32),
            pltpu.VMEM((PIECES, HP), jnp.int32),
            pltpu.VMEM((PIECES, HP), jnp.int32),
            pltpu.VMEM((PIECES, HP), jnp.int32),
            pltpu.VMEM((PIECES, HP), jnp.int32),
            pltpu.VMEM((BATCH_ROWS * H,), jnp.float32),
            pltpu.VMEM((PIECES * H,), jnp.float32),
            pltpu.VMEM((2 * 17 * L,), jnp.float32),
            pltpu.VMEM((2 * 17 * L,), jnp.float32),
            pltpu.VMEM((BATCH_ROWS, H), jnp.float32),
            pltpu.VMEM((BATCH_ROWS, H), jnp.float32),
            pltpu.VMEM((H,), jnp.float32),
            pltpu.VMEM((H,), jnp.float32),
            pltpu.SemaphoreType.DMA,
            pltpu.SemaphoreType.DMA,
            pltpu.SemaphoreType.DMA,
            pltpu.SemaphoreType.DMA,
        ],
    )
    out = sc(ids, wt_packed, pt_packed, ln_w, ln_b)
    return out.reshape(B, S, H)


# speedup vs baseline: 3.2348x; 1.2282x over previous
import jax
import jax.numpy as jnp
from jax import lax
from jax.experimental import pallas as pl
from jax.experimental.pallas import tpu as pltpu
from jax.experimental.pallas import tpu_sc as plsc

B, S, F = 16, 2048, 4
H = 768
HP = H // 2
VOCAB = 30522
MAX_POS = 8192
EPS = 1e-12

L = 16
KCH = H // L
PCH = HP // L
NW = 32
NROWS = B * S
ROWS_PER_W = NROWS // NW
BATCH_ROWS = 8
PIECES = BATCH_ROWS * F
NBATCH = ROWS_PER_W // BATCH_ROWS
CHUNK = ROWS_PER_W * F
SENT_PIECES = S * F


def _rsqrt(x):
    xi = lax.bitcast_convert_type(x, jnp.int32)
    yi = jnp.int32(0x5F3759DF) - lax.shift_right_logical(xi, 1)
    y = lax.bitcast_convert_type(yi, jnp.float32)
    for _ in range(4):
        y = y * (1.5 - 0.5 * x * y * y)
    return y


def _sc_body(ids_hbm, wt_hbm, pt_hbm, lnw_hbm, lnb_hbm, out_hbm,
             idsbuf, idxw0, idxp0, idxw1, idxp1,
             wbuf0, pbuf0, wbuf1, pbuf1,
             xbuf, sbuf, qbuf, outbuf0, outbuf1, lnwbuf, lnbbuf,
             gsem0, gsem1, osem0, osem1):
    wid = lax.axis_index("s") * 2 + lax.axis_index("c")
    sent = wid // 2
    half = wid % 2
    iota16 = lax.iota(jnp.int32, 16)
    zv = jnp.zeros((L,), jnp.float32)

    gslot = [(idxw0, idxp0, wbuf0, pbuf0, gsem0),
             (idxw1, idxp1, wbuf1, pbuf1, gsem1)]
    oslot = [(outbuf0, osem0), (outbuf1, osem1)]

    pltpu.sync_copy(lnw_hbm, lnwbuf)
    pltpu.sync_copy(lnb_hbm, lnbbuf)

    first_half_off = sent * SENT_PIECES
    pltpu.sync_copy(ids_hbm.at[pl.ds(first_half_off, CHUNK)], idsbuf)

    def _count_step(j, cnt):
        v = idsbuf[pl.ds(j * L, L)]
        return cnt + jnp.where(v != 0, 1, 0)

    cnt_v = lax.fori_loop(0, CHUNK // L, _count_step,
                          jnp.zeros((L,), jnp.int32), unroll=8)
    base0 = half * jnp.sum(cnt_v)

    my_off = wid * CHUNK
    pltpu.sync_copy(ids_hbm.at[pl.ds(my_off, CHUNK)], idsbuf)

    def _masks(i):
        off = i * PIECES
        ids0 = idsbuf[pl.ds(off, L)]
        ids1 = idsbuf[pl.ds(off + L, L)]
        return (jnp.where(ids0 != 0, 1, 0), jnp.where(ids1 != 0, 1, 0),
                ids0, ids1)

    def _fire(i, base, slot):
        idxw, idxp, wbuf, pbuf, gsem = gslot[slot]
        mi0, mi1, ids0, ids1 = _masks(i)
        c0 = plsc.cumsum(mi0)
        c1 = plsc.cumsum(mi1)
        t0 = jnp.sum(mi0)
        t1 = jnp.sum(mi1)
        pos0 = jnp.clip(base + c0 - 1, 0, MAX_POS - 1)
        pos1 = jnp.clip(base + t0 + c1 - 1, 0, MAX_POS - 1)
        idxw[pl.ds(0, L)] = ids0
        idxw[pl.ds(L, L)] = ids1
        idxp[pl.ds(0, L)] = pos0
        idxp[pl.ds(L, L)] = pos1
        pltpu.async_copy(wt_hbm.at[idxw], wbuf, gsem)
        pltpu.async_copy(pt_hbm.at[idxp], pbuf, gsem)
        return base + t0 + t1

    SQG = 17 * L

    def _splat(vec, lane):
        return jnp.take_along_axis(vec, jnp.full((L,), lane, jnp.int32),
                                   axis=0)

    def _unpk(v):
        return plsc.unpack(plsc.bitcast(v, jnp.bfloat16),
                           format=plsc.PackFormat.INTERLEAVED)

    def _process(i, slot):
        idxw, idxp, wbuf, pbuf, gsem = gslot[slot]
        outbuf, osem = oslot[slot]

        pltpu.make_async_copy(wt_hbm.at[idxw], wbuf, gsem).wait()
        pltpu.make_async_copy(pt_hbm.at[idxp], pbuf, gsem).wait()

        mi0, mi1, _, _ = _masks(i)

        iota17 = iota16 * 17

        def _stats(p, _):
            xoff = p * H
            GRP = 8

            def _stat(c8, carry):
                base = c8 * GRP
                wv = [wbuf[p, pl.ds((base + u) * L, L)] for u in range(GRP)]
                pv = [pbuf[p, pl.ds((base + u) * L, L)] for u in range(GRP)]
                accs = list(carry)
                for u in range(GRP):
                    wlo, whi = _unpk(wv[u])
                    plo, phi = _unpk(pv[u])
                    x0 = wlo + plo
                    x1 = whi + phi
                    xbuf[pl.ds(xoff + (base + u) * L, L)] = x0
                    xbuf[pl.ds(xoff + (base + u + PCH) * L, L)] = x1
                    j = u % 2
                    accs[j] = accs[j] + x0
                    accs[2 + j] = accs[2 + j] + x1
                    accs[4 + j] = accs[4 + j] + x0 * x0
                    accs[6 + j] = accs[6 + j] + x1 * x1
                return tuple(accs)

            acc = lax.fori_loop(0, PCH // GRP, _stat, (zv,) * 8)
            s_v = (acc[0] + acc[1]) + (acc[2] + acc[3])
            q_v = (acc[4] + acc[5]) + (acc[6] + acc[7])
            g = p // L
            off = iota17 + (p - g * L + g * SQG)
            plsc.store_scatter(sbuf, [off], s_v)
            plsc.store_scatter(qbuf, [off], q_v)
            return 0

        lax.fori_loop(0, PIECES, _stats, 0)

        perm1 = iota16 ^ 1
        perm2 = iota16 ^ 2
        rowfac = []
        for g, mi in ((0, mi0), (1, mi1)):
            tot_s = sbuf[pl.ds(g * SQG, L)]
            tot_q = qbuf[pl.ds(g * SQG, L)]
            for c in range(1, L):
                tot_s = tot_s + sbuf[pl.ds(g * SQG + c * 17, L)]
                tot_q = tot_q + qbuf[pl.ds(g * SQG + c * 17, L)]
            mu_v = tot_s * (1.0 / H)
            var_v = tot_q * (1.0 / H) - mu_v * mu_v
            rstd_v = _rsqrt(var_v + EPS)
            a_v = rstd_v * mi.astype(jnp.float32)
            b_v = -mu_v * a_v
            r1 = mi + jnp.take_along_axis(mi, perm1, axis=0)
            cnt4 = r1 + jnp.take_along_axis(r1, perm2, axis=0)
            inv_v = jnp.where(cnt4 <= 1, 1.0,
                              jnp.where(cnt4 == 2, 0.5,
                                        jnp.where(cnt4 == 3, 1.0 / 3.0,
                                                  0.25)))
            any_v = jnp.where(cnt4 > 0, 1.0, 0.0)
            bs1 = b_v + jnp.take_along_axis(b_v, perm1, axis=0)
            bsum_v = bs1 + jnp.take_along_axis(bs1, perm2, axis=0)
            rowfac.append((a_v * inv_v, bsum_v * inv_v, any_v))

        @pl.when(i >= 2)
        def _():
            pltpu.make_async_copy(
                outbuf, out_hbm.at[pl.ds(0, BATCH_ROWS)], osem).wait()

        for r in range(BATCH_ROWS):
            ap_v, bs_v, any_v = rowfac[0 if r < 4 else 1]
            lane = (r % 4) * F
            a_bc = [_splat(ap_v, lane + u) for u in range(F)]
            bs_bc = _splat(bs_v, lane)
            any_bc = _splat(any_v, lane)
            xbase = r * F * H

            GRP = 4

            def _fin(k4, _, r=r, a_bc=a_bc, bs_bc=bs_bc, any_bc=any_bc,
                     xbase=xbase):
                base = k4 * GRP
                xs = [[xbuf[pl.ds(xbase + u * H + (base + c) * L, L)]
                       for u in range(F)] for c in range(GRP)]
                lw = [lnwbuf[pl.ds((base + c) * L, L)] for c in range(GRP)]
                lb = [lnbbuf[pl.ds((base + c) * L, L)] for c in range(GRP)]
                for c in range(GRP):
                    t = xs[c][0] * a_bc[0]
                    for u in range(1, F):
                        t = t + xs[c][u] * a_bc[u]
                    o = (t + bs_bc) * lw[c] + lb[c] * any_bc
                    outbuf[r, pl.ds((base + c) * L, L)] = o
                return 0

            lax.fori_loop(0, KCH // GRP, _fin, 0)

        rowbase = wid * ROWS_PER_W + i * BATCH_ROWS
        pltpu.async_copy(outbuf, out_hbm.at[pl.ds(rowbase, BATCH_ROWS)], osem)

    base = _fire(0, base0, 0)

    def _pair(j, base):
        i0 = 2 * j
        base = _fire(i0 + 1, base, 1)
        _process(i0, 0)
        base = _fire(jnp.minimum(i0 + 2, NBATCH - 1), base, 0)
        _process(i0 + 1, 1)
        return base

    lax.fori_loop(0, NBATCH // 2, _pair, base)

    pltpu.make_async_copy(wt_hbm.at[idxw0], wbuf0, gsem0).wait()
    pltpu.make_async_copy(pt_hbm.at[idxp0], pbuf0, gsem0).wait()
    pltpu.make_async_copy(outbuf0, out_hbm.at[pl.ds(0, BATCH_ROWS)],
                          osem0).wait()
    pltpu.make_async_copy(outbuf1, out_hbm.at[pl.ds(0, BATCH_ROWS)],
                          osem1).wait()


def _pack_wt(x_ref, o_ref):
    e = x_ref[:, 0:HP]
    o = x_ref[:, HP:H]
    eu = lax.bitcast_convert_type(e.astype(jnp.bfloat16),
                                  jnp.uint16).astype(jnp.uint32)
    ou = lax.bitcast_convert_type(o.astype(jnp.bfloat16),
                                  jnp.uint16).astype(jnp.uint32)
    o_ref[...] = (eu | (ou << 16)).astype(jnp.int32)


def _pack_pt(pt_ref, tt_ref, o_ref):
    y = pt_ref[...] + tt_ref[...]
    e = y[:, 0:HP]
    o = y[:, HP:H]
    eu = lax.bitcast_convert_type(e.astype(jnp.bfloat16),
                                  jnp.uint16).astype(jnp.uint32)
    ou = lax.bitcast_convert_type(o.astype(jnp.bfloat16),
                                  jnp.uint16).astype(jnp.uint32)
    o_ref[...] = (eu | (ou << 16)).astype(jnp.int32)


def kernel(words, word_table, pos_table, tt_table, ln_w, ln_b):
    WBLK = 728
    wt_packed = pl.pallas_call(
        _pack_wt,
        grid=(42,),
        in_specs=[pl.BlockSpec((WBLK, H), lambda i: (i, 0))],
        out_specs=pl.BlockSpec((WBLK, HP), lambda i: (i, 0)),
        out_shape=jax.ShapeDtypeStruct((VOCAB, HP), jnp.int32),
    )(word_table)

    pt_packed = pl.pallas_call(
        _pack_pt,
        grid=(8,),
        in_specs=[
            pl.BlockSpec((MAX_POS // 8, H), lambda i: (i, 0)),
            pl.BlockSpec((1, H), lambda i: (0, 0)),
        ],
        out_specs=pl.BlockSpec((MAX_POS // 8, HP), lambda i: (i, 0)),
        out_shape=jax.ShapeDtypeStruct((MAX_POS, HP), jnp.int32),
    )(pos_table, tt_table[0:1])

    ids = words.reshape(NROWS * F)

    mesh = plsc.VectorSubcoreMesh(core_axis_name="c", subcore_axis_name="s")
    sc = pl.kernel(
        _sc_body,
        out_type=jax.ShapeDtypeStruct((NROWS, H), jnp.float32),
        mesh=mesh,
        compiler_params=pltpu.CompilerParams(needs_layout_passes=False),
        scratch_types=[
            pltpu.VMEM((CHUNK,), jnp.int32),
            pltpu.VMEM((PIECES,), jnp.int32),
            pltpu.VMEM((PIECES,), jnp.int32),
            pltpu.VMEM((PIECES,), jnp.int32),
            pltpu.VMEM((PIECES,), jnp.int32),
            pltpu.VMEM((PIECES, HP), jnp.int32),
            pltpu.VMEM((PIECES, HP), jnp.int32),
            pltpu.VMEM((PIECES, HP), jnp.int32),
            pltpu.VMEM((PIECES, HP), jnp.int32),
            pltpu.VMEM((PIECES * H,), jnp.float32),
            pltpu.VMEM((2 * 17 * L,), jnp.float32),
            pltpu.VMEM((2 * 17 * L,), jnp.float32),
            pltpu.VMEM((BATCH_ROWS, H), jnp.float32),
            pltpu.VMEM((BATCH_ROWS, H), jnp.float32),
            pltpu.VMEM((H,), jnp.float32),
            pltpu.VMEM((H,), jnp.float32),
            pltpu.SemaphoreType.DMA,
            pltpu.SemaphoreType.DMA,
            pltpu.SemaphoreType.DMA,
            pltpu.SemaphoreType.DMA,
        ],
    )
    out = sc(ids, wt_packed, pt_packed, ln_w, ln_b)
    return out.reshape(B, S, H)


# speedup vs baseline: 3.3252x; 1.0279x over previous
import jax
import jax.numpy as jnp
from jax import lax
from jax.experimental import pallas as pl
from jax.experimental.pallas import tpu as pltpu
from jax.experimental.pallas import tpu_sc as plsc

B, S, F = 16, 2048, 4
H = 768
HP = H // 2
VOCAB = 30522
MAX_POS = 8192
EPS = 1e-12

L = 16
KCH = H // L
PCH = HP // L
NW = 32
NROWS = B * S
ROWS_PER_W = NROWS // NW
BATCH_ROWS = 8
PIECES = BATCH_ROWS * F
NBATCH = ROWS_PER_W // BATCH_ROWS
CHUNK = ROWS_PER_W * F
SENT_PIECES = S * F


def _rsqrt(x):
    xi = lax.bitcast_convert_type(x, jnp.int32)
    yi = jnp.int32(0x5F3759DF) - lax.shift_right_logical(xi, 1)
    y = lax.bitcast_convert_type(yi, jnp.float32)
    for _ in range(4):
        y = y * (1.5 - 0.5 * x * y * y)
    return y


def _sc_body(ids_hbm, wt_hbm, pt_hbm, wsum_hbm, psum_hbm,
             lnw_hbm, lnb_hbm, out_hbm,
             idsbuf, idxw0, idxp0, idxw1, idxp1,
             wbuf0, pbuf0, wbuf1, pbuf1, wsbuf0, psbuf0, wsbuf1, psbuf1,
             xbuf, qbuf, outbuf0, outbuf1, lnwbuf, lnbbuf,
             gsem0, gsem1, osem0, osem1):
    wid = lax.axis_index("s") * 2 + lax.axis_index("c")
    sent = wid // 2
    half = wid % 2
    iota16 = lax.iota(jnp.int32, 16)
    zv = jnp.zeros((L,), jnp.float32)

    gslot = [(idxw0, idxp0, wbuf0, pbuf0, wsbuf0, psbuf0, gsem0),
             (idxw1, idxp1, wbuf1, pbuf1, wsbuf1, psbuf1, gsem1)]
    oslot = [(outbuf0, osem0), (outbuf1, osem1)]

    pltpu.sync_copy(lnw_hbm, lnwbuf)
    pltpu.sync_copy(lnb_hbm, lnbbuf)

    first_half_off = sent * SENT_PIECES
    pltpu.sync_copy(ids_hbm.at[pl.ds(first_half_off, CHUNK)], idsbuf)

    def _count_step(j, cnt):
        v = idsbuf[pl.ds(j * L, L)]
        return cnt + jnp.where(v != 0, 1, 0)

    cnt_v = lax.fori_loop(0, CHUNK // L, _count_step,
                          jnp.zeros((L,), jnp.int32), unroll=8)
    base0 = half * jnp.sum(cnt_v)

    my_off = wid * CHUNK
    pltpu.sync_copy(ids_hbm.at[pl.ds(my_off, CHUNK)], idsbuf)

    def _masks(i):
        off = i * PIECES
        ids0 = idsbuf[pl.ds(off, L)]
        ids1 = idsbuf[pl.ds(off + L, L)]
        return (jnp.where(ids0 != 0, 1, 0), jnp.where(ids1 != 0, 1, 0),
                ids0, ids1)

    def _fire(i, base, slot):
        idxw, idxp, wbuf, pbuf, wsbuf, psbuf, gsem = gslot[slot]
        mi0, mi1, ids0, ids1 = _masks(i)
        c0 = plsc.cumsum(mi0)
        c1 = plsc.cumsum(mi1)
        t0 = jnp.sum(mi0)
        t1 = jnp.sum(mi1)
        pos0 = jnp.clip(base + c0 - 1, 0, MAX_POS - 1)
        pos1 = jnp.clip(base + t0 + c1 - 1, 0, MAX_POS - 1)
        idxw[pl.ds(0, L)] = ids0
        idxw[pl.ds(L, L)] = ids1
        idxp[pl.ds(0, L)] = pos0
        idxp[pl.ds(L, L)] = pos1
        pltpu.async_copy(wt_hbm.at[idxw], wbuf, gsem)
        pltpu.async_copy(pt_hbm.at[idxp], pbuf, gsem)
        pltpu.async_copy(wsum_hbm.at[idxw], wsbuf, gsem)
        pltpu.async_copy(psum_hbm.at[idxp], psbuf, gsem)
        return base + t0 + t1

    SQG = 17 * L

    def _splat(vec, lane):
        return jnp.take_along_axis(vec, jnp.full((L,), lane, jnp.int32),
                                   axis=0)

    def _unpk(v):
        return plsc.unpack(plsc.bitcast(v, jnp.bfloat16),
                           format=plsc.PackFormat.INTERLEAVED)

    def _process(i, slot):
        idxw, idxp, wbuf, pbuf, wsbuf, psbuf, gsem = gslot[slot]
        outbuf, osem = oslot[slot]

        pltpu.make_async_copy(wt_hbm.at[idxw], wbuf, gsem).wait()
        pltpu.make_async_copy(pt_hbm.at[idxp], pbuf, gsem).wait()
        pltpu.make_async_copy(wsum_hbm.at[idxw], wsbuf, gsem).wait()
        pltpu.make_async_copy(psum_hbm.at[idxp], psbuf, gsem).wait()

        mi0, mi1, _, _ = _masks(i)

        iota17 = iota16 * 17

        def _stats(p, _):
            xoff = p * H
            GRP = 8

            def _stat(c8, carry):
                base = c8 * GRP
                wv = [wbuf[p, pl.ds((base + u) * L, L)] for u in range(GRP)]
                pv = [pbuf[p, pl.ds((base + u) * L, L)] for u in range(GRP)]
                accs = list(carry)
                for u in range(GRP):
                    wlo, whi = _unpk(wv[u])
                    plo, phi = _unpk(pv[u])
                    x0 = wlo + plo
                    x1 = whi + phi
                    xbuf[pl.ds(xoff + (base + u) * L, L)] = x0
                    xbuf[pl.ds(xoff + (base + u + PCH) * L, L)] = x1
                    j = u % 2
                    accs[j] = accs[j] + x0 * x0
                    accs[2 + j] = accs[2 + j] + x1 * x1
                return tuple(accs)

            acc = lax.fori_loop(0, PCH // GRP, _stat, (zv,) * 4)
            q_v = (acc[0] + acc[1]) + (acc[2] + acc[3])
            g = p // L
            off = iota17 + (p - g * L + g * SQG)
            plsc.store_scatter(qbuf, [off], q_v)
            return 0

        lax.fori_loop(0, PIECES, _stats, 0)

        perm1 = iota16 ^ 1
        perm2 = iota16 ^ 2
        rowfac = []
        for g, mi in ((0, mi0), (1, mi1)):
            tot_s = wsbuf[pl.ds(g * L, L)] + psbuf[pl.ds(g * L, L)]
            tot_q = qbuf[pl.ds(g * SQG, L)]
            for c in range(1, L):
                tot_q = tot_q + qbuf[pl.ds(g * SQG + c * 17, L)]
            mu_v = tot_s * (1.0 / H)
            var_v = tot_q * (1.0 / H) - mu_v * mu_v
            rstd_v = _rsqrt(var_v + EPS)
            a_v = rstd_v * mi.astype(jnp.float32)
            b_v = -mu_v * a_v
            r1 = mi + jnp.take_along_axis(mi, perm1, axis=0)
            cnt4 = r1 + jnp.take_along_axis(r1, perm2, axis=0)
            inv_v = jnp.where(cnt4 <= 1, 1.0,
                              jnp.where(cnt4 == 2, 0.5,
                                        jnp.where(cnt4 == 3, 1.0 / 3.0,
                                                  0.25)))
            any_v = jnp.where(cnt4 > 0, 1.0, 0.0)
            bs1 = b_v + jnp.take_along_axis(b_v, perm1, axis=0)
            bsum_v = bs1 + jnp.take_along_axis(bs1, perm2, axis=0)
            rowfac.append((a_v * inv_v, bsum_v * inv_v, any_v))

        @pl.when(i >= 2)
        def _():
            pltpu.make_async_copy(
                outbuf, out_hbm.at[pl.ds(0, BATCH_ROWS)], osem).wait()

        for r in range(BATCH_ROWS):
            ap_v, bs_v, any_v = rowfac[0 if r < 4 else 1]
            lane = (r % 4) * F
            a_bc = [_splat(ap_v, lane + u) for u in range(F)]
            bs_bc = _splat(bs_v, lane)
            any_bc = _splat(any_v, lane)
            xbase = r * F * H

            GRP = 4

            def _fin(k4, _, r=r, a_bc=a_bc, bs_bc=bs_bc, any_bc=any_bc,
                     xbase=xbase):
                base = k4 * GRP
                xs = [[xbuf[pl.ds(xbase + u * H + (base + c) * L, L)]
                       for u in range(F)] for c in range(GRP)]
                lw = [lnwbuf[pl.ds((base + c) * L, L)] for c in range(GRP)]
                lb = [lnbbuf[pl.ds((base + c) * L, L)] for c in range(GRP)]
                for c in range(GRP):
                    t = xs[c][0] * a_bc[0]
                    for u in range(1, F):
                        t = t + xs[c][u] * a_bc[u]
                    o = (t + bs_bc) * lw[c] + lb[c] * any_bc
                    outbuf[r, pl.ds((base + c) * L, L)] = o
                return 0

            lax.fori_loop(0, KCH // GRP, _fin, 0)

        rowbase = wid * ROWS_PER_W + i * BATCH_ROWS
        pltpu.async_copy(outbuf, out_hbm.at[pl.ds(rowbase, BATCH_ROWS)], osem)

    base = _fire(0, base0, 0)

    def _pair(j, base):
        i0 = 2 * j
        base = _fire(i0 + 1, base, 1)
        _process(i0, 0)
        base = _fire(jnp.minimum(i0 + 2, NBATCH - 1), base, 0)
        _process(i0 + 1, 1)
        return base

    lax.fori_loop(0, NBATCH // 2, _pair, base)

    pltpu.make_async_copy(wt_hbm.at[idxw0], wbuf0, gsem0).wait()
    pltpu.make_async_copy(pt_hbm.at[idxp0], pbuf0, gsem0).wait()
    pltpu.make_async_copy(wsum_hbm.at[idxw0], wsbuf0, gsem0).wait()
    pltpu.make_async_copy(psum_hbm.at[idxp0], psbuf0, gsem0).wait()
    pltpu.make_async_copy(outbuf0, out_hbm.at[pl.ds(0, BATCH_ROWS)],
                          osem0).wait()
    pltpu.make_async_copy(outbuf1, out_hbm.at[pl.ds(0, BATCH_ROWS)],
                          osem1).wait()


def _pack_wt(x_ref, o_ref, s_ref):
    e = x_ref[:, 0:HP].astype(jnp.bfloat16)
    o = x_ref[:, HP:H].astype(jnp.bfloat16)
    eu = lax.bitcast_convert_type(e, jnp.uint16).astype(jnp.uint32)
    ou = lax.bitcast_convert_type(o, jnp.uint16).astype(jnp.uint32)
    o_ref[...] = (eu | (ou << 16)).astype(jnp.int32)
    s_ref[...] = jnp.sum(e.astype(jnp.float32) + o.astype(jnp.float32),
                         axis=1).reshape(8, 128)


def _pack_pt(pt_ref, tt_ref, o_ref, s_ref):
    y = pt_ref[...] + tt_ref[...]
    e = y[:, 0:HP].astype(jnp.bfloat16)
    o = y[:, HP:H].astype(jnp.bfloat16)
    eu = lax.bitcast_convert_type(e, jnp.uint16).astype(jnp.uint32)
    ou = lax.bitcast_convert_type(o, jnp.uint16).astype(jnp.uint32)
    o_ref[...] = (eu | (ou << 16)).astype(jnp.int32)
    s_ref[...] = jnp.sum(e.astype(jnp.float32) + o.astype(jnp.float32),
                         axis=1).reshape(8, 128)


def kernel(words, word_table, pos_table, tt_table, ln_w, ln_b):
    WBLK = 1024
    WGRID = 30
    wt_packed, wsum2 = pl.pallas_call(
        _pack_wt,
        grid=(WGRID,),
        in_specs=[pl.BlockSpec((WBLK, H), lambda i: (i, 0))],
        out_specs=[pl.BlockSpec((WBLK, HP), lambda i: (i, 0)),
                   pl.BlockSpec((8, 128), lambda i: (i, 0))],
        out_shape=[jax.ShapeDtypeStruct((WGRID * WBLK, HP), jnp.int32),
                   jax.ShapeDtypeStruct((WGRID * 8, 128), jnp.float32)],
    )(word_table)
    wsum = wsum2.reshape(WGRID * WBLK)

    PBLK = MAX_POS // 8
    pt_packed, psum2 = pl.pallas_call(
        _pack_pt,
        grid=(8,),
        in_specs=[
            pl.BlockSpec((PBLK, H), lambda i: (i, 0)),
            pl.BlockSpec((1, H), lambda i: (0, 0)),
        ],
        out_specs=[pl.BlockSpec((PBLK, HP), lambda i: (i, 0)),
                   pl.BlockSpec((8, 128), lambda i: (i, 0))],
        out_shape=[jax.ShapeDtypeStruct((MAX_POS, HP), jnp.int32),
                   jax.ShapeDtypeStruct((64, 128), jnp.float32)],
    )(pos_table, tt_table[0:1])
    psum = psum2.reshape(MAX_POS)

    ids = words.reshape(NROWS * F)

    mesh = plsc.VectorSubcoreMesh(core_axis_name="c", subcore_axis_name="s")
    sc = pl.kernel(
        _sc_body,
        out_type=jax.ShapeDtypeStruct((NROWS, H), jnp.float32),
        mesh=mesh,
        compiler_params=pltpu.CompilerParams(needs_layout_passes=False),
        scratch_types=[
            pltpu.VMEM((CHUNK,), jnp.int32),
            pltpu.VMEM((PIECES,), jnp.int32),
            pltpu.VMEM((PIECES,), jnp.int32),
            pltpu.VMEM((PIECES,), jnp.int32),
            pltpu.VMEM((PIECES,), jnp.int32),
            pltpu.VMEM((PIECES, HP), jnp.int32),
            pltpu.VMEM((PIECES, HP), jnp.int32),
            pltpu.VMEM((PIECES, HP), jnp.int32),
            pltpu.VMEM((PIECES, HP), jnp.int32),
            pltpu.VMEM((PIECES,), jnp.float32),
            pltpu.VMEM((PIECES,), jnp.float32),
            pltpu.VMEM((PIECES,), jnp.float32),
            pltpu.VMEM((PIECES,), jnp.float32),
            pltpu.VMEM((PIECES * H,), jnp.float32),
            pltpu.VMEM((2 * 17 * L,), jnp.float32),
            pltpu.VMEM((BATCH_ROWS, H), jnp.float32),
            pltpu.VMEM((BATCH_ROWS, H), jnp.float32),
            pltpu.VMEM((H,), jnp.float32),
            pltpu.VMEM((H,), jnp.float32),
            pltpu.SemaphoreType.DMA,
            pltpu.SemaphoreType.DMA,
            pltpu.SemaphoreType.DMA,
            pltpu.SemaphoreType.DMA,
        ],
    )
    out = sc(ids, wt_packed, pt_packed, wsum, psum, ln_w, ln_b)
    return out.reshape(B, S, H)
